# Initial kernel scaffold; baseline (speedup 1.0000x reference)
import functools

import jax
import jax.numpy as jnp
from jax import lax
from jax.experimental import pallas as pl
from jax.experimental.pallas import tpu as pltpu
from jax.experimental.pallas import tpu_sc as plsc

N = 10000
E = 320000
D = 128
G = 64

NC = 2
NS = 16
NW = NC * NS
CH = 128
NCHUNK = E // CH
FULL = NCHUNK // NW
LEFT = NCHUNK - FULL * NW
RPW = N // NS
ZR = 125


def _sc_agg_body(h_hbm, src_hbm, dst_hbm, out_hbm,
                 gbuf, sidx, didx, zbuf, acc, sem):
  c = lax.axis_index("c")
  s = lax.axis_index("s")
  wid = c * NS + s

  zv = jnp.zeros((16,), jnp.float32)
  def zero_row(i, _):
    for k in range(D // 16):
      zbuf[i, pl.ds(k * 16, 16)] = zv
    return 0
  lax.fori_loop(0, ZR, zero_row, 0)
  for r in range(RPW // ZR):
    pltpu.sync_copy(zbuf, acc.at[pl.ds(s * RPW + r * ZR, ZR)])
  plsc.subcore_barrier()

  def do_chunk(cid):
    pltpu.sync_copy(src_hbm.at[cid], sidx.at[0])
    pltpu.sync_copy(dst_hbm.at[cid], didx.at[0])
    pltpu.async_copy(h_hbm.at[sidx.at[0]], gbuf.at[0], sem).wait()
    pltpu.sync_copy(gbuf.at[0], acc.at[didx.at[0]], add=True)

  def chunk_body(j, _):
    do_chunk(j * NW + wid)
    return 0
  lax.fori_loop(0, FULL, chunk_body, 0)

  @pl.when(wid < LEFT)
  def _():
    do_chunk(FULL * NW + wid)

  plsc.subcore_barrier()
  pltpu.sync_copy(acc.at[pl.ds(s * RPW, RPW)],
                  out_hbm.at[c, pl.ds(s * RPW, RPW)])


@jax.jit
def _sc_aggregate(h, src2, dst2):
  mesh = plsc.VectorSubcoreMesh(core_axis_name="c", subcore_axis_name="s")
  kern = pl.kernel(
      _sc_agg_body,
      out_type=jax.ShapeDtypeStruct((NC, N, D), jnp.float32),
      mesh=mesh,
      scratch_types=[
          pltpu.VMEM((1, CH, D), jnp.float32),
          pltpu.VMEM((1, CH), jnp.int32),
          pltpu.VMEM((1, CH), jnp.int32),
          pltpu.VMEM((ZR, D), jnp.float32),
          pltpu.VMEM_SHARED((N, D), jnp.float32),
          pltpu.SemaphoreType.DMA,
      ],
  )
  return kern(h, src2, dst2)


def _dense_body(acc_ref, h_ref, wr_ref, b_ref, wo_ref, o_ref):
  a = acc_ref[0] + acc_ref[1]
  z = lax.dot_general(a, wr_ref[...], (((1,), (1,)), ((), ())),
                      preferred_element_type=jnp.float32)
  z = z + lax.dot_general(h_ref[...], wo_ref[...], (((1,), (1,)), ((), ())),
                          preferred_element_type=jnp.float32)
  o_ref[...] = jnp.maximum(z + b_ref[...], 0.0)


BR = 2000


@jax.jit
def _tc_dense(acc, h, wr, b, wo):
  return pl.pallas_call(
      _dense_body,
      grid=(N // BR,),
      in_specs=[
          pl.BlockSpec((NC, BR, D), lambda i: (0, i, 0)),
          pl.BlockSpec((BR, D), lambda i: (i, 0)),
          pl.BlockSpec((D, D), lambda i: (0, 0)),
          pl.BlockSpec((1, D), lambda i: (0, 0)),
          pl.BlockSpec((D, D), lambda i: (0, 0)),
      ],
      out_specs=pl.BlockSpec((BR, D), lambda i: (i, 0)),
      out_shape=jax.ShapeDtypeStruct((N, D), jnp.float32),
  )(acc, h, wr, b.reshape(1, D), wo)


def _final_body(acc_ref, h_ref, wr_ref, b_ref, wo_ref, batch_ref,
                f1w_ref, f1b_ref, f2w_ref, f2b_ref, o_ref, pool):
  i = pl.program_id(0)
  a = acc_ref[0] + acc_ref[1]
  z = lax.dot_general(a, wr_ref[...], (((1,), (1,)), ((), ())),
                      preferred_element_type=jnp.float32)
  z = z + lax.dot_general(h_ref[...], wo_ref[...], (((1,), (1,)), ((), ())),
                          preferred_element_type=jnp.float32)
  z = jnp.maximum(z + b_ref[...], 0.0)
  gids = lax.broadcasted_iota(jnp.int32, (1, G), 1)
  oh = (batch_ref[...] == gids).astype(jnp.float32)
  p = lax.dot_general(oh, z, (((0,), (0,)), ((), ())),
                      preferred_element_type=jnp.float32)

  @pl.when(i == 0)
  def _():
    pool[...] = p

  @pl.when(i > 0)
  def _():
    pool[...] = pool[...] + p

  @pl.when(i == N // BR - 1)
  def _():
    q = lax.dot_general(pool[...], f1w_ref[...], (((1,), (1,)), ((), ())),
                        preferred_element_type=jnp.float32)
    q = jnp.maximum(q + f1b_ref[...], 0.0)
    o = lax.dot_general(q, f2w_ref[...], (((1,), (1,)), ((), ())),
                        preferred_element_type=jnp.float32)
    o_ref[...] = o + f2b_ref[...]


@jax.jit
def _tc_final(acc, h, wr, b, wo, batch2, f1w, f1b, f2w, f2b):
  return pl.pallas_call(
      _final_body,
      grid=(N // BR,),
      in_specs=[
          pl.BlockSpec((NC, BR, D), lambda i: (0, i, 0)),
          pl.BlockSpec((BR, D), lambda i: (i, 0)),
          pl.BlockSpec((D, D), lambda i: (0, 0)),
          pl.BlockSpec((1, D), lambda i: (0, 0)),
          pl.BlockSpec((D, D), lambda i: (0, 0)),
          pl.BlockSpec((BR, 1), lambda i: (i, 0)),
          pl.BlockSpec((D, D), lambda i: (0, 0)),
          pl.BlockSpec((1, D), lambda i: (0, 0)),
          pl.BlockSpec((1, D), lambda i: (0, 0)),
          pl.BlockSpec((1, 1), lambda i: (0, 0)),
      ],
      out_specs=pl.BlockSpec((G, 1), lambda i: (0, 0)),
      out_shape=jax.ShapeDtypeStruct((G, 1), jnp.float32),
      scratch_shapes=[pltpu.VMEM((G, D), jnp.float32)],
  )(acc, h, wr, b.reshape(1, D), wo, batch2,
    f1w, f1b.reshape(1, D), f2w.reshape(1, D), f2b.reshape(1, 1))


def kernel(x, edge_index, batch, w1_rel, b1_rel, w1_root, w2_rel, b2_rel,
           w2_root, w3_rel, b3_rel, w3_root, fc1_w, fc1_b, fc2_w, fc2_b):
  ei = edge_index.astype(jnp.int32)
  src2 = ei[0].reshape(NCHUNK, CH)
  dst2 = ei[1].reshape(NCHUNK, CH)
  batch2 = batch.astype(jnp.int32).reshape(N, 1)

  acc = _sc_aggregate(x, src2, dst2)
  h1 = _tc_dense(acc, x, w1_rel, b1_rel, w1_root)
  acc = _sc_aggregate(h1, src2, dst2)
  h2 = _tc_dense(acc, h1, w2_rel, b2_rel, w2_root)
  acc = _sc_aggregate(h2, src2, dst2)
  return _tc_final(acc, h2, w3_rel, b3_rel, w3_root, batch2,
                   fc1_w, fc1_b, fc2_w, fc2_b)


# speedup vs baseline: 6.4541x; 6.4541x over previous
import functools

import jax
import jax.numpy as jnp
from jax import lax
from jax.experimental import pallas as pl
from jax.experimental.pallas import tpu as pltpu
from jax.experimental.pallas import tpu_sc as plsc

N = 10000
E = 320000
D = 128
G = 64

NC = 2
NS = 16
NW = NC * NS
CH = 128
NCHUNK = E // CH
FULL = NCHUNK // NW
LEFT = NCHUNK - FULL * NW
RB = 624
TAIL = N - RB * NS
ZR = 208


def _sc_agg_body(h_hbm, src_hbm, dst_hbm, out_hbm,
                 gbuf, sidx, didx, zbuf, acc, sem):
  c = lax.axis_index("c")
  s = lax.axis_index("s")
  wid = c * NS + s

  zv = jnp.zeros((16,), jnp.float32)
  def zero_row(i, _):
    for k in range(D // 16):
      zbuf[i, pl.ds(k * 16, 16)] = zv
    return 0
  lax.fori_loop(0, ZR, zero_row, 0)
  for r in range(RB // ZR):
    pltpu.sync_copy(zbuf, acc.at[pl.ds(s * RB + r * ZR, ZR)])

  @pl.when(s == 0)
  def _():
    pltpu.sync_copy(zbuf.at[pl.ds(0, TAIL)], acc.at[pl.ds(NS * RB, TAIL)])

  plsc.subcore_barrier()

  def do_chunk(cid):
    pltpu.sync_copy(src_hbm.at[cid], sidx.at[0])
    pltpu.sync_copy(dst_hbm.at[cid], didx.at[0])
    pltpu.async_copy(h_hbm.at[sidx.at[0]], gbuf.at[0], sem).wait()
    pltpu.sync_copy(gbuf.at[0], acc.at[didx.at[0]], add=True)

  def chunk_body(j, _):
    do_chunk(j * NW + wid)
    return 0
  lax.fori_loop(0, FULL, chunk_body, 0)

  @pl.when(wid < LEFT)
  def _():
    do_chunk(FULL * NW + wid)

  plsc.subcore_barrier()
  pltpu.sync_copy(acc.at[pl.ds(s * RB, RB)],
                  out_hbm.at[c, pl.ds(s * RB, RB)])

  @pl.when(s == 0)
  def _():
    pltpu.sync_copy(acc.at[pl.ds(NS * RB, TAIL)],
                    out_hbm.at[c, pl.ds(NS * RB, TAIL)])


@jax.jit
def _sc_aggregate(h, src2, dst2):
  mesh = plsc.VectorSubcoreMesh(core_axis_name="c", subcore_axis_name="s")
  kern = pl.kernel(
      _sc_agg_body,
      out_type=jax.ShapeDtypeStruct((NC, N, D), jnp.float32),
      mesh=mesh,
      scratch_types=[
          pltpu.VMEM((1, CH, D), jnp.float32),
          pltpu.VMEM((1, CH), jnp.int32),
          pltpu.VMEM((1, CH), jnp.int32),
          pltpu.VMEM((ZR, D), jnp.float32),
          pltpu.VMEM_SHARED((N, D), jnp.float32),
          pltpu.SemaphoreType.DMA,
      ],
  )
  return kern(h, src2, dst2)


def _dense_body(acc_ref, h_ref, wr_ref, b_ref, wo_ref, o_ref):
  a = acc_ref[0] + acc_ref[1]
  z = lax.dot_general(a, wr_ref[...], (((1,), (1,)), ((), ())),
                      preferred_element_type=jnp.float32)
  z = z + lax.dot_general(h_ref[...], wo_ref[...], (((1,), (1,)), ((), ())),
                          preferred_element_type=jnp.float32)
  o_ref[...] = jnp.maximum(z + b_ref[...], 0.0)


BR = 2000


@jax.jit
def _tc_dense(acc, h, wr, b, wo):
  return pl.pallas_call(
      _dense_body,
      grid=(N // BR,),
      in_specs=[
          pl.BlockSpec((NC, BR, D), lambda i: (0, i, 0)),
          pl.BlockSpec((BR, D), lambda i: (i, 0)),
          pl.BlockSpec((D, D), lambda i: (0, 0)),
          pl.BlockSpec((1, D), lambda i: (0, 0)),
          pl.BlockSpec((D, D), lambda i: (0, 0)),
      ],
      out_specs=pl.BlockSpec((BR, D), lambda i: (i, 0)),
      out_shape=jax.ShapeDtypeStruct((N, D), jnp.float32),
  )(acc, h, wr, b.reshape(1, D), wo)


def _final_body(acc_ref, h_ref, wr_ref, b_ref, wo_ref, batch_ref,
                f1w_ref, f1b_ref, f2w_ref, f2b_ref, o_ref, pool):
  i = pl.program_id(0)
  a = acc_ref[0] + acc_ref[1]
  z = lax.dot_general(a, wr_ref[...], (((1,), (1,)), ((), ())),
                      preferred_element_type=jnp.float32)
  z = z + lax.dot_general(h_ref[...], wo_ref[...], (((1,), (1,)), ((), ())),
                          preferred_element_type=jnp.float32)
  z = jnp.maximum(z + b_ref[...], 0.0)
  gids = lax.broadcasted_iota(jnp.int32, (1, G), 1)
  oh = (batch_ref[...] == gids).astype(jnp.float32)
  p = lax.dot_general(oh, z, (((0,), (0,)), ((), ())),
                      preferred_element_type=jnp.float32)

  @pl.when(i == 0)
  def _():
    pool[...] = p

  @pl.when(i > 0)
  def _():
    pool[...] = pool[...] + p

  @pl.when(i == N // BR - 1)
  def _():
    q = lax.dot_general(pool[...], f1w_ref[...], (((1,), (1,)), ((), ())),
                        preferred_element_type=jnp.float32)
    q = jnp.maximum(q + f1b_ref[...], 0.0)
    o = jnp.sum(q * f2w_ref[...], axis=1, keepdims=True)
    o_ref[...] = o + f2b_ref[0, 0]


@jax.jit
def _tc_final(acc, h, wr, b, wo, batch2, f1w, f1b, f2w, f2b):
  return pl.pallas_call(
      _final_body,
      grid=(N // BR,),
      in_specs=[
          pl.BlockSpec((NC, BR, D), lambda i: (0, i, 0)),
          pl.BlockSpec((BR, D), lambda i: (i, 0)),
          pl.BlockSpec((D, D), lambda i: (0, 0)),
          pl.BlockSpec((1, D), lambda i: (0, 0)),
          pl.BlockSpec((D, D), lambda i: (0, 0)),
          pl.BlockSpec((BR, 1), lambda i: (i, 0)),
          pl.BlockSpec((D, D), lambda i: (0, 0)),
          pl.BlockSpec((1, D), lambda i: (0, 0)),
          pl.BlockSpec((1, D), lambda i: (0, 0)),
          pl.BlockSpec((1, 1), lambda i: (0, 0)),
      ],
      out_specs=pl.BlockSpec((G, 1), lambda i: (0, 0)),
      out_shape=jax.ShapeDtypeStruct((G, 1), jnp.float32),
      scratch_shapes=[pltpu.VMEM((G, D), jnp.float32)],
  )(acc, h, wr, b.reshape(1, D), wo, batch2,
    f1w, f1b.reshape(1, D), f2w.reshape(1, D), f2b.reshape(1, 1))


def kernel(x, edge_index, batch, w1_rel, b1_rel, w1_root, w2_rel, b2_rel,
           w2_root, w3_rel, b3_rel, w3_root, fc1_w, fc1_b, fc2_w, fc2_b):
  ei = edge_index.astype(jnp.int32)
  src2 = ei[0].reshape(NCHUNK, CH)
  dst2 = ei[1].reshape(NCHUNK, CH)
  batch2 = batch.astype(jnp.int32).reshape(N, 1)

  acc = _sc_aggregate(x, src2, dst2)
  h1 = _tc_dense(acc, x, w1_rel, b1_rel, w1_root)
  acc = _sc_aggregate(h1, src2, dst2)
  h2 = _tc_dense(acc, h1, w2_rel, b2_rel, w2_root)
  acc = _sc_aggregate(h2, src2, dst2)
  return _tc_final(acc, h2, w3_rel, b3_rel, w3_root, batch2,
                   fc1_w, fc1_b, fc2_w, fc2_b)


# speedup vs baseline: 13.2444x; 2.0521x over previous
import functools

import jax
import jax.numpy as jnp
from jax import lax
from jax.experimental import pallas as pl
from jax.experimental.pallas import tpu as pltpu
from jax.experimental.pallas import tpu_sc as plsc

N = 10000
E = 320000
D = 128
G = 64

NC = 2
NS = 16
NW = NC * NS
CH = 128
CPT = 80
NCHUNK = NW * CPT
E_PAD = NCHUNK * CH
REAL = E // CH
SEG = 40
RB = 624
TAIL = N - RB * NS


def _sc_agg_body(h_hbm, src_hbm, dst_hbm, out_hbm,
                 gbuf, sidx, didx, acc, sem0, sem1):
  c = lax.axis_index("c")
  s = lax.axis_index("s")
  wid = c * NS + s

  zv = jnp.zeros((16,), jnp.float32)
  def zero_row(i, _):
    for k in range(D // 16):
      gbuf[0, i, pl.ds(k * 16, 16)] = zv
    return 0
  lax.fori_loop(0, CH, zero_row, 0)
  for r in range(4):
    pltpu.sync_copy(gbuf.at[0], acc.at[pl.ds(s * RB + r * CH, CH)])
  pltpu.sync_copy(gbuf.at[0, pl.ds(0, RB - 4 * CH)],
                  acc.at[pl.ds(s * RB + 4 * CH, RB - 4 * CH)])

  @pl.when(s == 0)
  def _():
    pltpu.sync_copy(gbuf.at[0, pl.ds(0, TAIL)], acc.at[pl.ds(NS * RB, TAIL)])

  plsc.subcore_barrier()

  sems = (sem0, sem1)

  def gather(l, slot):
    pltpu.async_copy(h_hbm.at[sidx.at[l]], gbuf.at[slot], sems[slot])

  def scatter(l, slot):
    pltpu.make_async_copy(h_hbm.at[sidx.at[l]], gbuf.at[slot],
                          sems[slot]).wait()
    pltpu.sync_copy(gbuf.at[slot], acc.at[didx.at[l]], add=True)

  def run_seg(njj, seg):
    pltpu.sync_copy(src_hbm.at[pl.ds(wid * CPT + seg * SEG, SEG)], sidx)
    pltpu.sync_copy(dst_hbm.at[pl.ds(wid * CPT + seg * SEG, SEG)], didx)

    @pl.when(njj > 0)
    def _():
      gather(0, 0)

    def body2(jj, _):
      l0 = 2 * jj
      gather(l0 + 1, 1)
      scatter(l0, 0)

      @pl.when(jj < njj - 1)
      def _():
        gather(l0 + 2, 0)

      scatter(l0 + 1, 1)
      return 0
    lax.fori_loop(0, njj, body2, 0)

  t31 = wid == NW - 1
  run_seg(jnp.where(t31, (REAL - (NW - 1) * CPT) // 2, SEG // 2), 0)
  run_seg(jnp.where(t31, 0, SEG // 2), 1)

  plsc.subcore_barrier()
  pltpu.sync_copy(acc.at[pl.ds(s * RB, RB)],
                  out_hbm.at[c, pl.ds(s * RB, RB)])

  @pl.when(s == 0)
  def _():
    pltpu.sync_copy(acc.at[pl.ds(NS * RB, TAIL)],
                    out_hbm.at[c, pl.ds(NS * RB, TAIL)])


@jax.jit
def _sc_aggregate(h, src2, dst2):
  mesh = plsc.VectorSubcoreMesh(core_axis_name="c", subcore_axis_name="s")
  kern = pl.kernel(
      _sc_agg_body,
      out_type=jax.ShapeDtypeStruct((NC, N, D), jnp.float32),
      mesh=mesh,
      scratch_types=[
          pltpu.VMEM((2, CH, D), jnp.float32),
          pltpu.VMEM((SEG, CH), jnp.int32),
          pltpu.VMEM((SEG, CH), jnp.int32),
          pltpu.VMEM_SHARED((N, D), jnp.float32),
          pltpu.SemaphoreType.DMA,
          pltpu.SemaphoreType.DMA,
      ],
  )
  return kern(h, src2, dst2)


def _dense_body(acc_ref, h_ref, wr_ref, b_ref, wo_ref, o_ref):
  a = acc_ref[0] + acc_ref[1]
  z = lax.dot_general(a, wr_ref[...], (((1,), (1,)), ((), ())),
                      preferred_element_type=jnp.float32)
  z = z + lax.dot_general(h_ref[...], wo_ref[...], (((1,), (1,)), ((), ())),
                          preferred_element_type=jnp.float32)
  o_ref[...] = jnp.maximum(z + b_ref[...], 0.0)


BR = 2000


@jax.jit
def _tc_dense(acc, h, wr, b, wo):
  return pl.pallas_call(
      _dense_body,
      grid=(N // BR,),
      in_specs=[
          pl.BlockSpec((NC, BR, D), lambda i: (0, i, 0)),
          pl.BlockSpec((BR, D), lambda i: (i, 0)),
          pl.BlockSpec((D, D), lambda i: (0, 0)),
          pl.BlockSpec((1, D), lambda i: (0, 0)),
          pl.BlockSpec((D, D), lambda i: (0, 0)),
      ],
      out_specs=pl.BlockSpec((BR, D), lambda i: (i, 0)),
      out_shape=jax.ShapeDtypeStruct((N, D), jnp.float32),
  )(acc, h, wr, b.reshape(1, D), wo)


def _final_body(acc_ref, h_ref, wr_ref, b_ref, wo_ref, batch_ref,
                f1w_ref, f1b_ref, f2w_ref, f2b_ref, o_ref, pool):
  i = pl.program_id(0)
  a = acc_ref[0] + acc_ref[1]
  z = lax.dot_general(a, wr_ref[...], (((1,), (1,)), ((), ())),
                      preferred_element_type=jnp.float32)
  z = z + lax.dot_general(h_ref[...], wo_ref[...], (((1,), (1,)), ((), ())),
                          preferred_element_type=jnp.float32)
  z = jnp.maximum(z + b_ref[...], 0.0)
  gids = lax.broadcasted_iota(jnp.int32, (1, G), 1)
  oh = (batch_ref[...] == gids).astype(jnp.float32)
  p = lax.dot_general(oh, z, (((0,), (0,)), ((), ())),
                      preferred_element_type=jnp.float32)

  @pl.when(i == 0)
  def _():
    pool[...] = p

  @pl.when(i > 0)
  def _():
    pool[...] = pool[...] + p

  @pl.when(i == N // BR - 1)
  def _():
    q = lax.dot_general(pool[...], f1w_ref[...], (((1,), (1,)), ((), ())),
                        preferred_element_type=jnp.float32)
    q = jnp.maximum(q + f1b_ref[...], 0.0)
    o = jnp.sum(q * f2w_ref[...], axis=1, keepdims=True)
    o_ref[...] = o + f2b_ref[0, 0]


@jax.jit
def _tc_final(acc, h, wr, b, wo, batch2, f1w, f1b, f2w, f2b):
  return pl.pallas_call(
      _final_body,
      grid=(N // BR,),
      in_specs=[
          pl.BlockSpec((NC, BR, D), lambda i: (0, i, 0)),
          pl.BlockSpec((BR, D), lambda i: (i, 0)),
          pl.BlockSpec((D, D), lambda i: (0, 0)),
          pl.BlockSpec((1, D), lambda i: (0, 0)),
          pl.BlockSpec((D, D), lambda i: (0, 0)),
          pl.BlockSpec((BR, 1), lambda i: (i, 0)),
          pl.BlockSpec((D, D), lambda i: (0, 0)),
          pl.BlockSpec((1, D), lambda i: (0, 0)),
          pl.BlockSpec((1, D), lambda i: (0, 0)),
          pl.BlockSpec((1, 1), lambda i: (0, 0)),
      ],
      out_specs=pl.BlockSpec((G, 1), lambda i: (0, 0)),
      out_shape=jax.ShapeDtypeStruct((G, 1), jnp.float32),
      scratch_shapes=[pltpu.VMEM((G, D), jnp.float32)],
  )(acc, h, wr, b.reshape(1, D), wo, batch2,
    f1w, f1b.reshape(1, D), f2w.reshape(1, D), f2b.reshape(1, 1))


def kernel(x, edge_index, batch, w1_rel, b1_rel, w1_root, w2_rel, b2_rel,
           w2_root, w3_rel, b3_rel, w3_root, fc1_w, fc1_b, fc2_w, fc2_b):
  ei = edge_index.astype(jnp.int32)
  pad = jnp.zeros((E_PAD - E,), jnp.int32)
  src2 = jnp.concatenate([ei[0], pad]).reshape(NCHUNK, CH)
  dst2 = jnp.concatenate([ei[1], pad]).reshape(NCHUNK, CH)
  batch2 = batch.astype(jnp.int32).reshape(N, 1)

  acc = _sc_aggregate(x, src2, dst2)
  h1 = _tc_dense(acc, x, w1_rel, b1_rel, w1_root)
  acc = _sc_aggregate(h1, src2, dst2)
  h2 = _tc_dense(acc, h1, w2_rel, b2_rel, w2_root)
  acc = _sc_aggregate(h2, src2, dst2)
  return _tc_final(acc, h2, w3_rel, b3_rel, w3_root, batch2,
                   fc1_w, fc1_b, fc2_w, fc2_b)
